# baseline (speedup 1.0000x reference)
import jax
import jax.numpy as jnp
from jax.experimental import pallas as pl
from jax.experimental.pallas import tpu as pltpu
from functools import partial

_EPS = 1.1920929e-07
_HB = 8
_NSLOT = 3
_NSPLIT = 4


def _axis_pool_kernel(x_hbm, cw_ref, nw_ref, o_ref, buf, sem, *, d, nb, ng):
    nsteps = nb * ng
    dh = d // _NSPLIT
    cw = cw_ref[...]
    nw = nw_ref[...]

    def copies(i, slot):
        b = i // ng
        g = jax.lax.rem(i, ng)
        rows = pl.ds(g * _HB, _HB)
        return tuple(
            pltpu.make_async_copy(
                x_hbm.at[b, pl.ds(q * dh, dh), rows, :],
                buf.at[slot, pl.ds(q * dh, dh)],
                sem.at[slot, q],
            )
            for q in range(_NSPLIT)
        )

    def start(i, slot):
        for c in copies(i, slot):
            c.start()

    def wait(i, slot):
        for c in copies(i, slot):
            c.wait()

    start(0, 0)
    start(1, 1)

    def body(i, carry):
        cur = jax.lax.rem(i, _NSLOT)
        nxt = jax.lax.rem(i + 2, _NSLOT)

        @pl.when(i + 2 < nsteps)
        def _():
            start(i + 2, nxt)

        cops = copies(i, cur)
        c = None
        t = None
        for q in range(_NSPLIT):
            cops[q].wait()
            Xq = buf[cur, q * dh:(q + 1) * dh]
            cq = jnp.sum(Xq * Xq, axis=0)
            tq = jnp.sum(Xq * cw[q * dh:(q + 1) * dh], axis=0)
            c = cq if c is None else c + cq
            t = tq if t is None else t + tq
        X = buf[cur]
        r = jax.lax.rsqrt(c * (1.0 / d) + _EPS)
        s = t * r
        m = jnp.max(s, axis=1, keepdims=True)
        e = jnp.exp(s - m)
        denom = jnp.sum(e, axis=1, keepdims=True)
        g2 = e * (r / denom)
        pooled = jnp.sum(X * g2[None], axis=2)
        b = i // ng
        g = jax.lax.rem(i, ng)
        o_ref[b, pl.ds(g * _HB, _HB), :] = pooled.T * nw
        return 0

    jax.lax.fori_loop(0, nsteps, body, 0)


def kernel(x, norm_weight, score_weight):
    b, d, h, w = x.shape
    ng = h // _HB
    cw = (norm_weight * score_weight).reshape(d, 1, 1)
    nw = norm_weight.reshape(1, d)
    return pl.pallas_call(
        partial(_axis_pool_kernel, d=d, nb=b, ng=ng),
        out_shape=jax.ShapeDtypeStruct((b, h, d), x.dtype),
        in_specs=[
            pl.BlockSpec(memory_space=pl.ANY),
            pl.BlockSpec(memory_space=pltpu.VMEM),
            pl.BlockSpec(memory_space=pltpu.VMEM),
        ],
        out_specs=pl.BlockSpec(memory_space=pltpu.VMEM),
        scratch_shapes=[
            pltpu.VMEM((_NSLOT, d, _HB, w), jnp.float32),
            pltpu.SemaphoreType.DMA((_NSLOT, _NSPLIT)),
        ],
        compiler_params=pltpu.CompilerParams(
            vmem_limit_bytes=56 * 1024 * 1024,
        ),
        name="axis_attn_pool",
    )(x, cw, nw)


# speedup vs baseline: 1.0103x; 1.0103x over previous
import jax
import jax.numpy as jnp
from jax.experimental import pallas as pl
from jax.experimental.pallas import tpu as pltpu
from functools import partial

_EPS = 1.1920929e-07
_HB = 8
_NSLOT = 3
_NSPLIT = 4


def _axis_pool_kernel(x_hbm, cw_ref, nw_ref, o_ref, buf, sem, *, d, nb, ng):
    nsteps = nb * ng
    dh = d // _NSPLIT
    cw = cw_ref[...]
    nw = nw_ref[...]

    def copies(i, slot):
        b = i // ng
        g = jax.lax.rem(i, ng)
        rows = pl.ds(g * _HB, _HB)
        return tuple(
            pltpu.make_async_copy(
                x_hbm.at[b, pl.ds(q * dh, dh), rows, :],
                buf.at[slot, pl.ds(q * dh, dh)],
                sem.at[slot, q],
            )
            for q in range(_NSPLIT)
        )

    def start(i, slot):
        for c in copies(i, slot):
            c.start()

    def wait(i, slot):
        for c in copies(i, slot):
            c.wait()

    start(0, 0)
    start(1, 1)

    def body(i, carry):
        cur = jax.lax.rem(i, _NSLOT)
        nxt = jax.lax.rem(i + 2, _NSLOT)

        @pl.when(i + 2 < nsteps)
        def _():
            start(i + 2, nxt)

        wait(i, cur)
        X = buf[cur]
        c = jnp.sum(X * X, axis=0)
        t = jnp.sum(X * cw, axis=0)
        r = jax.lax.rsqrt(c * (1.0 / d) + _EPS)
        s = t * r
        m = jnp.max(s, axis=1, keepdims=True)
        e = jnp.exp(s - m)
        denom = jnp.sum(e, axis=1, keepdims=True)
        g2 = e * (r / denom)
        pooled = jnp.sum(X * g2[None], axis=2)
        b = i // ng
        g = jax.lax.rem(i, ng)
        o_ref[b, pl.ds(g * _HB, _HB), :] = pooled.T * nw
        return 0

    jax.lax.fori_loop(0, nsteps, body, 0)


def kernel(x, norm_weight, score_weight):
    b, d, h, w = x.shape
    ng = h // _HB
    cw = (norm_weight * score_weight).reshape(d, 1, 1)
    nw = norm_weight.reshape(1, d)
    return pl.pallas_call(
        partial(_axis_pool_kernel, d=d, nb=b, ng=ng),
        out_shape=jax.ShapeDtypeStruct((b, h, d), x.dtype),
        in_specs=[
            pl.BlockSpec(memory_space=pl.ANY),
            pl.BlockSpec(memory_space=pltpu.VMEM),
            pl.BlockSpec(memory_space=pltpu.VMEM),
        ],
        out_specs=pl.BlockSpec(memory_space=pltpu.VMEM),
        scratch_shapes=[
            pltpu.VMEM((_NSLOT, d, _HB, w), jnp.float32),
            pltpu.SemaphoreType.DMA((_NSLOT, _NSPLIT)),
        ],
        compiler_params=pltpu.CompilerParams(
            vmem_limit_bytes=56 * 1024 * 1024,
        ),
        name="axis_attn_pool",
    )(x, cw, nw)
